# baseline (speedup 1.0000x reference)
import functools

import jax
import jax.numpy as jnp
from jax import lax
from jax.experimental import pallas as pl
from jax.experimental.pallas import tpu as pltpu
from jax.experimental.pallas import tpu_sc as plsc

N_DET = 5000
N_PAD = 5120
N_OBJ = 64
NW = 16
DPW = N_PAD // NW
CH = DPW // 16
THRESHOLD = 0.5
NEG_POS_RATIO = 3
BIG_I = 1 << 30


def _sc_body(db_hbm, ds_hbm, bt_hbm, lab_hbm, out_hbm,
             db_v, ds_v, bt_v, lab_v, ov_v, obj_v, rmax_v, ridx_v,
             detobj_v, conf_v, pack_v, allrm_v, allri_v, allconf_v, sums_v,
             sh_rmax, sh_ridx, sh_conf, sh_sums):
    cid = lax.axis_index("c")
    w = lax.axis_index("s")
    base = w * DPW
    iota16 = lax.broadcasted_iota(jnp.int32, (16,), 0)

    def _shuf(x, s):
        return x.at[iota16 ^ s].get(mode="promise_in_bounds")

    def _allred(x, op):
        for s in (1, 2, 4, 8):
            x = op(x, _shuf(x, s))
        return x

    pltpu.sync_copy(db_hbm.at[w], db_v)
    pltpu.sync_copy(ds_hbm.at[w], ds_v)
    pltpu.sync_copy(bt_hbm, bt_v)
    pltpu.sync_copy(lab_hbm, lab_v)

    for k in range(CH):
        ov_v[k] = jnp.full((16,), -1.0, jnp.float32)
        obj_v[k] = jnp.zeros((16,), jnp.int32)

    def obj_body(g, j, carry):
        mvec, dvec = carry
        i = g * 16 + j
        bx1 = bt_v[0, pl.ds(i, 16)][0]
        by1 = bt_v[1, pl.ds(i, 16)][0]
        bx2 = bt_v[2, pl.ds(i, 16)][0]
        by2 = bt_v[3, pl.ds(i, 16)][0]
        a1 = (bx2 - bx1) * (by2 - by1)
        lmax = jnp.full((16,), -2.0, jnp.float32)
        lidx = jnp.zeros((16,), jnp.int32)
        for k in range(CH):
            dx1 = db_v[0, pl.ds(16 * k, 16)]
            dy1 = db_v[1, pl.ds(16 * k, 16)]
            dx2 = db_v[2, pl.ds(16 * k, 16)]
            dy2 = db_v[3, pl.ds(16 * k, 16)]
            iw = jnp.maximum(jnp.minimum(bx2, dx2) - jnp.maximum(bx1, dx1),
                             0.0)
            ih = jnp.maximum(jnp.minimum(by2, dy2) - jnp.maximum(by1, dy1),
                             0.0)
            inter = iw * ih
            a2 = (dx2 - dx1) * (dy2 - dy1)
            iou = inter / (a1 + a2 - inter)
            gidx = base + 16 * k + iota16
            iou = jnp.where(gidx < N_DET, iou, -1.0)
            ovk = ov_v[k]
            upd = iou > ovk
            ov_v[k] = jnp.where(upd, iou, ovk)
            obj_v[k] = jnp.where(upd, i, obj_v[k])
            lupd = iou > lmax
            lmax = jnp.where(lupd, iou, lmax)
            lidx = jnp.where(lupd, gidx, lidx)
        m = _allred(lmax, jnp.maximum)
        cand = jnp.where(lmax == m, lidx, BIG_I)
        dglob = _allred(cand, jnp.minimum)
        mvec = jnp.where(iota16 == j, m, mvec)
        dvec = jnp.where(iota16 == j, dglob, dvec)
        return mvec, dvec

    for g in range(4):
        mvec, dvec = lax.fori_loop(
            0, 16, functools.partial(obj_body, g),
            (jnp.full((16,), -2.0, jnp.float32), jnp.zeros((16,), jnp.int32)))
        rmax_v[pl.ds(g * 16, 16)] = mvec
        ridx_v[pl.ds(g * 16, 16)] = dvec

    pltpu.sync_copy(rmax_v, sh_rmax.at[pl.ds(w * 64, 64)])
    pltpu.sync_copy(ridx_v, sh_ridx.at[pl.ds(w * 64, 64)])
    plsc.subcore_barrier()

    pltpu.sync_copy(sh_rmax, allrm_v)
    pltpu.sync_copy(sh_ridx, allri_v)
    for g in range(4):
        m = jnp.full((16,), -2.0, jnp.float32)
        for ww in range(NW):
            m = jnp.maximum(m, allrm_v[pl.ds((ww * 4 + g) * 16, 16)])
        d = jnp.full((16,), BIG_I, jnp.int32)
        for ww in range(NW):
            rmw = allrm_v[pl.ds((ww * 4 + g) * 16, 16)]
            riw = allri_v[pl.ds((ww * 4 + g) * 16, 16)]
            d = jnp.where(rmw == m, jnp.minimum(d, riw), d)
        detobj_v[pl.ds(g * 16, 16)] = d

    for k in range(CH):
        gidx = base + 16 * k + iota16

        def scat_body(i, carry, k=k, gidx=gidx):
            ovr, obr = carry
            dd = detobj_v[pl.ds(i, 16)][0]
            eq = gidx == dd
            return jnp.where(eq, 1.0, ovr), jnp.where(eq, i, obr)

        ovr, obr = lax.fori_loop(0, N_OBJ, scat_body, (ov_v[k], obj_v[k]))
        ov_v[k] = ovr
        obj_v[k] = obr

    lab_t = [lab_v[pl.ds(16 * q, 16)] for q in range(4)]
    bt_t = [[bt_v[c, pl.ds(16 * q, 16)] for q in range(4)] for c in range(4)]

    def _tbl(regs, idx):
        q = jnp.right_shift(idx, 4)
        low = jnp.bitwise_and(idx, 15)
        g = [r.at[low].get(mode="promise_in_bounds") for r in regs]
        return jnp.where(q == 0, g[0],
                         jnp.where(q == 1, g[1],
                                   jnp.where(q == 2, g[2], g[3])))

    n_pos = jnp.zeros((16,), jnp.float32)
    conf_pos = jnp.zeros((16,), jnp.float32)
    loc_sum = jnp.zeros((16,), jnp.float32)
    for k in range(CH):
        ovr = ov_v[k]
        obr = obj_v[k]
        labd = _tbl(lab_t, obr)
        labd = jnp.where(ovr < THRESHOLD, 0, labd)
        posm = labd != 0
        posf = posm.astype(jnp.float32)
        n_pos = n_pos + posf
        conf = 1.0 - ds_v[pl.ds(16 * k, 16)]
        conf_pos = conf_pos + conf * posf
        conf_v[pl.ds(16 * k, 16)] = jnp.where(posm, 0.0, conf)
        acc = jnp.abs(db_v[0, pl.ds(16 * k, 16)] - _tbl(bt_t[0], obr))
        acc = acc + jnp.abs(db_v[1, pl.ds(16 * k, 16)] - _tbl(bt_t[1], obr))
        acc = acc + jnp.abs(db_v[2, pl.ds(16 * k, 16)] - _tbl(bt_t[2], obr))
        acc = acc + jnp.abs(db_v[3, pl.ds(16 * k, 16)] - _tbl(bt_t[3], obr))
        loc_sum = loc_sum + acc * posf

    pack_v[...] = (jnp.where(iota16 == 0, _allred(n_pos, jnp.add), 0.0)
                   + jnp.where(iota16 == 1, _allred(conf_pos, jnp.add), 0.0)
                   + jnp.where(iota16 == 2, _allred(loc_sum, jnp.add), 0.0))
    pltpu.sync_copy(pack_v, sh_sums.at[pl.ds(w * 16, 16)])
    pltpu.sync_copy(conf_v, sh_conf.at[pl.ds(w * DPW, DPW)])
    plsc.subcore_barrier()

    @pl.when(w == 0)
    def _final():
        pltpu.sync_copy(sh_sums, sums_v)
        pltpu.sync_copy(sh_conf, allconf_v)
        tot = jnp.zeros((16,), jnp.float32)
        for ww in range(NW):
            tot = tot + sums_v[pl.ds(ww * 16, 16)]
        def _bcast(x, lane):
            return x.at[jnp.full((16,), lane, jnp.int32)].get(
                mode="promise_in_bounds")

        n_pos_b = _bcast(tot, 0)
        conf_pos_b = _bcast(tot, 1)
        loc_b = _bcast(tot, 2)
        kn_b = 3.0 * n_pos_b
        kn = kn_b[0]

        def count_ge(mid):
            cnt = jnp.zeros((16,), jnp.float32)
            for ww in range(NW):
                for k in range(CH):
                    bits = plsc.bitcast(
                        allconf_v[pl.ds((ww * CH + k) * 16, 16)], jnp.int32)
                    cnt = cnt + (bits >= mid).astype(jnp.float32)
            return _allred(cnt, jnp.add)[0]

        def search(_, carry):
            lo, hi = carry
            mid = lax.div(lo + hi, 2)
            ok = count_ge(mid) >= kn
            return jnp.where(ok, mid, lo), jnp.where(ok, hi, mid)

        lo, _ = lax.fori_loop(0, 31, search,
                              (jnp.int32(0), jnp.int32(0x3F800001)))
        tf_b = plsc.bitcast(jnp.full((16,), lo, jnp.int32), jnp.float32)
        cnt_gt = jnp.zeros((16,), jnp.float32)
        sum_gt = jnp.zeros((16,), jnp.float32)
        for ww in range(NW):
            for k in range(CH):
                cv = allconf_v[pl.ds((ww * CH + k) * 16, 16)]
                gt = plsc.bitcast(cv, jnp.int32) > lo
                cnt_gt = cnt_gt + gt.astype(jnp.float32)
                sum_gt = sum_gt + jnp.where(gt, cv, 0.0)
        conf_hard_b = (_allred(sum_gt, jnp.add)
                       + (kn_b - _allred(cnt_gt, jnp.add)) * tf_b)
        total_b = ((conf_hard_b + conf_pos_b) / n_pos_b
                   + loc_b / (n_pos_b * 4.0))
        pack_v[...] = jnp.where(iota16 == 0, total_b, 0.0)

        @pl.when(cid == 0)
        def _write():
            pltpu.sync_copy(pack_v, out_hbm)


@jax.jit
def kernel(det_boxes, det_scores, det_labels, boxes, labels):
    del det_labels
    db = jnp.pad(det_boxes[0].astype(jnp.float32),
                 ((0, N_PAD - N_DET), (0, 0)))
    db = db.reshape(NW, DPW, 4).transpose(0, 2, 1)
    ds = jnp.pad(det_scores[0].astype(jnp.float32), (0, N_PAD - N_DET),
                 constant_values=1.0).reshape(NW, DPW)
    bt = jnp.pad(boxes[0].astype(jnp.float32).T, ((0, 0), (0, 16)))
    lab = labels[0].astype(jnp.int32)

    mesh = plsc.VectorSubcoreMesh(core_axis_name="c", subcore_axis_name="s")
    run = functools.partial(
        pl.kernel, _sc_body, mesh=mesh,
        out_type=jax.ShapeDtypeStruct((16,), jnp.float32),
        compiler_params=pltpu.CompilerParams(needs_layout_passes=False),
        scratch_types=[
            pltpu.VMEM((4, DPW), jnp.float32),
            pltpu.VMEM((DPW,), jnp.float32),
            pltpu.VMEM((4, N_OBJ + 16), jnp.float32),
            pltpu.VMEM((N_OBJ,), jnp.int32),
            pltpu.VMEM((CH, 16), jnp.float32),
            pltpu.VMEM((CH, 16), jnp.int32),
            pltpu.VMEM((N_OBJ,), jnp.float32),
            pltpu.VMEM((N_OBJ,), jnp.int32),
            pltpu.VMEM((N_OBJ + 16,), jnp.int32),
            pltpu.VMEM((DPW,), jnp.float32),
            pltpu.VMEM((16,), jnp.float32),
            pltpu.VMEM((NW * N_OBJ,), jnp.float32),
            pltpu.VMEM((NW * N_OBJ,), jnp.int32),
            pltpu.VMEM((N_PAD,), jnp.float32),
            pltpu.VMEM((NW * 16,), jnp.float32),
            pltpu.VMEM_SHARED((NW * N_OBJ,), jnp.float32),
            pltpu.VMEM_SHARED((NW * N_OBJ,), jnp.int32),
            pltpu.VMEM_SHARED((N_PAD,), jnp.float32),
            pltpu.VMEM_SHARED((NW * 16,), jnp.float32),
        ],
    )
    out = run()(db, ds, bt, lab)
    return out[0]
